# Initial kernel scaffold; baseline (speedup 1.0000x reference)
import jax
import jax.numpy as jnp
from jax.experimental import pallas as pl


def kernel(x, inref_y, inref_x):
    raise NotImplementedError("write your pallas kernel here")


# speedup vs baseline: 1015.6537x; 1015.6537x over previous
import jax
import jax.numpy as jnp
from jax import lax
from jax.experimental import pallas as pl
from jax.experimental.pallas import tpu as pltpu
from jax.experimental.pallas import tpu_sc as plsc

B, C, H, W = 4, 32, 128, 128
HW = H * W
K = 9
M = 9 * 64 * 64
J = K * M
CH = 4096
NCHUNK = M // CH
LANES = 16
TPC = 4
NW = 32
WPB = NW // B
UNROLL = 4


def _gather_body(xf, yf, xif, out, t0, t1, t2, t3, yb, xb, outb):
    tables = (t0, t1, t2, t3)
    cid = lax.axis_index("c")
    sid = lax.axis_index("s")
    wid = sid * 2 + cid
    b = wid // WPB
    cbase = (wid % WPB) * TPC

    for j in range(TPC):
        pltpu.sync_copy(xf.at[b, cbase + j], tables[j])

    def k_loop(k, carry):
        def t_loop(t, carry):
            joff = k * M + t * CH
            pltpu.sync_copy(yf.at[b, pl.ds(joff, CH)], yb)
            pltpu.sync_copy(xif.at[b, pl.ds(joff, CH)], xb)

            def i_loop(i, carry):
                base = i * (LANES * UNROLL)
                for u in range(UNROLL):
                    off = base + u * LANES
                    yv = yb[pl.ds(off, LANES)]
                    xv = xb[pl.ds(off, LANES)]
                    iv = xv + yv * W
                    for j in range(TPC):
                        outb[j, pl.ds(off, LANES)] = plsc.load_gather(
                            tables[j], [iv])
                return carry

            lax.fori_loop(0, CH // (LANES * UNROLL), i_loop, 0)
            for j in range(TPC):
                pltpu.sync_copy(outb.at[j],
                                out.at[b, k, cbase + j, pl.ds(t * CH, CH)])
            return carry

        return lax.fori_loop(0, NCHUNK, t_loop, carry)

    lax.fori_loop(0, K, k_loop, 0)


@jax.jit
def kernel(x, inref_y, inref_x):
    xf = x.reshape(B, C, HW)
    yf = inref_y.reshape(B, J)
    xif = inref_x.reshape(B, J)
    mesh = plsc.VectorSubcoreMesh(core_axis_name="c", subcore_axis_name="s")
    out = pl.kernel(
        _gather_body,
        out_type=jax.ShapeDtypeStruct((B, K, C, M), jnp.float32),
        mesh=mesh,
        compiler_params=pltpu.CompilerParams(needs_layout_passes=False),
        scratch_types=[
            pltpu.VMEM((HW,), jnp.float32),
            pltpu.VMEM((HW,), jnp.float32),
            pltpu.VMEM((HW,), jnp.float32),
            pltpu.VMEM((HW,), jnp.float32),
            pltpu.VMEM((CH,), jnp.int32),
            pltpu.VMEM((CH,), jnp.int32),
            pltpu.VMEM((TPC, CH), jnp.float32),
        ],
    )(xf, yf, xif)
    return out


# speedup vs baseline: 1347.6335x; 1.3269x over previous
import jax
import jax.numpy as jnp
from jax import lax
from jax.experimental import pallas as pl
from jax.experimental.pallas import tpu as pltpu
from jax.experimental.pallas import tpu_sc as plsc

B, C, H, W = 4, 32, 128, 128
HW = H * W
K = 9
M = 9 * 64 * 64
J = K * M
CH = 4096
NCHUNK = M // CH
NTOT = K * NCHUNK
LANES = 16
TPC = 4
NW = 32
WPB = NW // B
UNROLL = 4


def _gather_body(xf, yf, xif, out, t0, t1, t2, t3, ybuf, xbuf, obuf,
                 sl0, sl1, ss0, ss1):
    tables = (t0, t1, t2, t3)
    semld = (sl0, sl1)
    semst = (ss0, ss1)
    cid = lax.axis_index("c")
    sid = lax.axis_index("s")
    wid = sid * 2 + cid
    b = wid // WPB
    cbase = (wid % WPB) * TPC

    for j in range(TPC):
        pltpu.sync_copy(xf.at[b, cbase + j], tables[j])

    def ld(n, q):
        pltpu.async_copy(yf.at[b, pl.ds(n * CH, CH)], ybuf.at[q], semld[q])
        pltpu.async_copy(xif.at[b, pl.ds(n * CH, CH)], xbuf.at[q], semld[q])

    def ld_wait(n, q):
        pltpu.make_async_copy(
            yf.at[b, pl.ds(n * CH, CH)], ybuf.at[q], semld[q]).wait()
        pltpu.make_async_copy(
            xif.at[b, pl.ds(n * CH, CH)], xbuf.at[q], semld[q]).wait()

    def out_slice(n):
        k = n // NCHUNK
        t = n % NCHUNK
        return out.at[b, k, pl.ds(cbase, TPC), pl.ds(t * CH, CH)]

    def st(n, q):
        pltpu.async_copy(obuf.at[q], out_slice(n), semst[q])

    def st_wait(n, q):
        pltpu.make_async_copy(obuf.at[q], out_slice(n), semst[q]).wait()

    def compute(q):
        def i_loop(i, carry):
            base = i * (LANES * UNROLL)
            for u in range(UNROLL):
                off = base + u * LANES
                yv = ybuf[q, pl.ds(off, LANES)]
                xv = xbuf[q, pl.ds(off, LANES)]
                iv = xv + yv * W
                for j in range(TPC):
                    obuf[q, j, pl.ds(off, LANES)] = plsc.load_gather(
                        tables[j], [iv])
            return carry

        lax.fori_loop(0, CH // (LANES * UNROLL), i_loop, 0)

    ld(0, 0)
    ld(1, 1)
    ld_wait(0, 0)
    compute(0)
    ld(2, 0)
    st(0, 0)
    ld_wait(1, 1)
    compute(1)
    ld(3, 1)
    st(1, 1)

    def pair(p, carry):
        for q in (0, 1):
            n = 2 * p + q
            ld_wait(n, q)
            st_wait(n - 2, q)
            compute(q)

            @pl.when(n + 2 <= NTOT - 1)
            def _():
                ld(n + 2, q)

            st(n, q)
        return carry

    lax.fori_loop(1, (NTOT - 1) // 2, pair, 0)

    ld_wait(NTOT - 1, 0)
    st_wait(NTOT - 3, 0)
    compute(0)
    st(NTOT - 1, 0)
    st_wait(NTOT - 2, 1)
    st_wait(NTOT - 1, 0)


@jax.jit
def kernel(x, inref_y, inref_x):
    xf = x.reshape(B, C, HW)
    yf = inref_y.reshape(B, J)
    xif = inref_x.reshape(B, J)
    mesh = plsc.VectorSubcoreMesh(core_axis_name="c", subcore_axis_name="s")
    out = pl.kernel(
        _gather_body,
        out_type=jax.ShapeDtypeStruct((B, K, C, M), jnp.float32),
        mesh=mesh,
        compiler_params=pltpu.CompilerParams(needs_layout_passes=False),
        scratch_types=[
            pltpu.VMEM((HW,), jnp.float32),
            pltpu.VMEM((HW,), jnp.float32),
            pltpu.VMEM((HW,), jnp.float32),
            pltpu.VMEM((HW,), jnp.float32),
            pltpu.VMEM((2, CH), jnp.int32),
            pltpu.VMEM((2, CH), jnp.int32),
            pltpu.VMEM((2, TPC, CH), jnp.float32),
            pltpu.SemaphoreType.DMA,
            pltpu.SemaphoreType.DMA,
            pltpu.SemaphoreType.DMA,
            pltpu.SemaphoreType.DMA,
        ],
    )(xf, yf, xif)
    return out


# speedup vs baseline: 4340.5084x; 3.2208x over previous
import jax
import jax.numpy as jnp
from jax import lax
from jax.experimental import pallas as pl
from jax.experimental.pallas import tpu as pltpu
from jax.experimental.pallas import tpu_sc as plsc

B, C, H, W = 4, 32, 128, 128
HW = H * W
K = 9
M = 9 * 64 * 64
J = K * M
CH = 4096
NCHUNK = M // CH
NTOT = K * NCHUNK
LANES = 16
TPC = 4
NW = 32
WPB = NW // B
UNROLL = 4


def _gather_body(xf, yf, xif, out, t0, t1, t2, t3, ybuf, xbuf, obuf,
                 sl0, sl1, ss0, ss1):
    tables = (t0, t1, t2, t3)
    semld = (sl0, sl1)
    semst = (ss0, ss1)
    cid = lax.axis_index("c")
    sid = lax.axis_index("s")
    wid = sid * 2 + cid
    b = wid // WPB
    cbase = (wid % WPB) * TPC

    for j in range(TPC):
        pltpu.sync_copy(xf.at[b, cbase + j], tables[j])

    def ld(n, q):
        pltpu.async_copy(yf.at[b, pl.ds(n * CH, CH)], ybuf.at[q], semld[q])
        pltpu.async_copy(xif.at[b, pl.ds(n * CH, CH)], xbuf.at[q], semld[q])

    def ld_wait(n, q):
        pltpu.make_async_copy(
            yf.at[b, pl.ds(n * CH, CH)], ybuf.at[q], semld[q]).wait()
        pltpu.make_async_copy(
            xif.at[b, pl.ds(n * CH, CH)], xbuf.at[q], semld[q]).wait()

    def out_slice(n):
        k = n // NCHUNK
        t = n % NCHUNK
        return out.at[b, k, pl.ds(cbase, TPC), pl.ds(t * CH, CH)]

    def st(n, q):
        pltpu.async_copy(obuf.at[q], out_slice(n), semst[q])

    def st_wait(n, q):
        pltpu.make_async_copy(obuf.at[q], out_slice(n), semst[q]).wait()

    def compute(q):
        @plsc.parallel_loop(0, CH, LANES, unroll=UNROLL)
        def _(off):
            yv = ybuf[q, pl.ds(off, LANES)]
            xv = xbuf[q, pl.ds(off, LANES)]
            iv = xv + yv * W
            for j in range(TPC):
                obuf[q, j, pl.ds(off, LANES)] = plsc.load_gather(
                    tables[j], [iv])

    ld(0, 0)
    ld(1, 1)
    ld_wait(0, 0)
    compute(0)
    ld(2, 0)
    st(0, 0)
    ld_wait(1, 1)
    compute(1)
    ld(3, 1)
    st(1, 1)

    def pair(p, carry):
        for q in (0, 1):
            n = 2 * p + q
            ld_wait(n, q)
            st_wait(n - 2, q)
            compute(q)

            @pl.when(n + 2 <= NTOT - 1)
            def _():
                ld(n + 2, q)

            st(n, q)
        return carry

    lax.fori_loop(1, (NTOT - 1) // 2, pair, 0)

    ld_wait(NTOT - 1, 0)
    st_wait(NTOT - 3, 0)
    compute(0)
    st(NTOT - 1, 0)
    st_wait(NTOT - 2, 1)
    st_wait(NTOT - 1, 0)


@jax.jit
def kernel(x, inref_y, inref_x):
    xf = x.reshape(B, C, HW)
    yf = inref_y.reshape(B, J)
    xif = inref_x.reshape(B, J)
    mesh = plsc.VectorSubcoreMesh(core_axis_name="c", subcore_axis_name="s")
    out = pl.kernel(
        _gather_body,
        out_type=jax.ShapeDtypeStruct((B, K, C, M), jnp.float32),
        mesh=mesh,
        compiler_params=pltpu.CompilerParams(needs_layout_passes=False),
        scratch_types=[
            pltpu.VMEM((HW,), jnp.float32),
            pltpu.VMEM((HW,), jnp.float32),
            pltpu.VMEM((HW,), jnp.float32),
            pltpu.VMEM((HW,), jnp.float32),
            pltpu.VMEM((2, CH), jnp.int32),
            pltpu.VMEM((2, CH), jnp.int32),
            pltpu.VMEM((2, TPC, CH), jnp.float32),
            pltpu.SemaphoreType.DMA,
            pltpu.SemaphoreType.DMA,
            pltpu.SemaphoreType.DMA,
            pltpu.SemaphoreType.DMA,
        ],
    )(xf, yf, xif)
    return out


# speedup vs baseline: 4358.2494x; 1.0041x over previous
import jax
import jax.numpy as jnp
from jax import lax
from jax.experimental import pallas as pl
from jax.experimental.pallas import tpu as pltpu
from jax.experimental.pallas import tpu_sc as plsc

B, C, H, W = 4, 32, 128, 128
HW = H * W
K = 9
M = 9 * 64 * 64
J = K * M
CH = 4096
NCHUNK = M // CH
NTOT = K * NCHUNK
LANES = 16
TPC = 4
NW = 32
WPB = NW // B
UNROLL = 8


def _gather_body(xf, yf, xif, out, t0, t1, t2, t3, ybuf, xbuf, obuf,
                 sl0, sl1, ss0, ss1):
    tables = (t0, t1, t2, t3)
    semld = (sl0, sl1)
    semst = (ss0, ss1)
    cid = lax.axis_index("c")
    sid = lax.axis_index("s")
    wid = sid * 2 + cid
    b = wid // WPB
    cbase = (wid % WPB) * TPC

    for j in range(TPC):
        pltpu.sync_copy(xf.at[b, cbase + j], tables[j])

    def ld(n, q):
        pltpu.async_copy(yf.at[b, pl.ds(n * CH, CH)], ybuf.at[q], semld[q])
        pltpu.async_copy(xif.at[b, pl.ds(n * CH, CH)], xbuf.at[q], semld[q])

    def ld_wait(n, q):
        pltpu.make_async_copy(
            yf.at[b, pl.ds(n * CH, CH)], ybuf.at[q], semld[q]).wait()
        pltpu.make_async_copy(
            xif.at[b, pl.ds(n * CH, CH)], xbuf.at[q], semld[q]).wait()

    def out_slice(n):
        k = n // NCHUNK
        t = n % NCHUNK
        return out.at[b, k, pl.ds(cbase, TPC), pl.ds(t * CH, CH)]

    def st(n, q):
        pltpu.async_copy(obuf.at[q], out_slice(n), semst[q])

    def st_wait(n, q):
        pltpu.make_async_copy(obuf.at[q], out_slice(n), semst[q]).wait()

    def compute(q):
        @plsc.parallel_loop(0, CH, LANES, unroll=UNROLL)
        def _(off):
            yv = ybuf[q, pl.ds(off, LANES)]
            xv = xbuf[q, pl.ds(off, LANES)]
            iv = xv + yv * W
            for j in range(TPC):
                obuf[q, j, pl.ds(off, LANES)] = plsc.load_gather(
                    tables[j], [iv])

    ld(0, 0)
    ld(1, 1)
    ld_wait(0, 0)
    compute(0)
    ld(2, 0)
    st(0, 0)
    ld_wait(1, 1)
    compute(1)
    ld(3, 1)
    st(1, 1)

    def pair(p, carry):
        for q in (0, 1):
            n = 2 * p + q
            ld_wait(n, q)
            st_wait(n - 2, q)
            compute(q)

            @pl.when(n + 2 <= NTOT - 1)
            def _():
                ld(n + 2, q)

            st(n, q)
        return carry

    lax.fori_loop(1, (NTOT - 1) // 2, pair, 0)

    ld_wait(NTOT - 1, 0)
    st_wait(NTOT - 3, 0)
    compute(0)
    st(NTOT - 1, 0)
    st_wait(NTOT - 2, 1)
    st_wait(NTOT - 1, 0)


@jax.jit
def kernel(x, inref_y, inref_x):
    xf = x.reshape(B, C, HW)
    yf = inref_y.reshape(B, J)
    xif = inref_x.reshape(B, J)
    mesh = plsc.VectorSubcoreMesh(core_axis_name="c", subcore_axis_name="s")
    out = pl.kernel(
        _gather_body,
        out_type=jax.ShapeDtypeStruct((B, K, C, M), jnp.float32),
        mesh=mesh,
        compiler_params=pltpu.CompilerParams(needs_layout_passes=False),
        scratch_types=[
            pltpu.VMEM((HW,), jnp.float32),
            pltpu.VMEM((HW,), jnp.float32),
            pltpu.VMEM((HW,), jnp.float32),
            pltpu.VMEM((HW,), jnp.float32),
            pltpu.VMEM((2, CH), jnp.int32),
            pltpu.VMEM((2, CH), jnp.int32),
            pltpu.VMEM((2, TPC, CH), jnp.float32),
            pltpu.SemaphoreType.DMA,
            pltpu.SemaphoreType.DMA,
            pltpu.SemaphoreType.DMA,
            pltpu.SemaphoreType.DMA,
        ],
    )(xf, yf, xif)
    return out


# speedup vs baseline: 4798.1582x; 1.1009x over previous
import jax
import jax.numpy as jnp
from jax import lax
from jax.experimental import pallas as pl
from jax.experimental.pallas import tpu as pltpu
from jax.experimental.pallas import tpu_sc as plsc

B, C, H, W = 4, 32, 128, 128
HW = H * W
K = 9
M = 9 * 64 * 64
J = K * M
CH = 4096
NCHUNK = M // CH
NTOT = K * NCHUNK
LANES = 16
TPC = 4
NW = 32
WPB = NW // B
UNROLL = 8

FTOT = B * J
FPW = FTOT // NW
FCH = 4608
FNC = FPW // FCH


def _flatten_body(yf, xif, ifl, ybuf, xbuf, obuf, sl0, sl1, ss0, ss1):
    semld = (sl0, sl1)
    semst = (ss0, ss1)
    cid = lax.axis_index("c")
    sid = lax.axis_index("s")
    wid = sid * 2 + cid

    def ld(n, q):
        pltpu.async_copy(yf.at[wid, pl.ds(n * FCH, FCH)], ybuf.at[q],
                         semld[q])
        pltpu.async_copy(xif.at[wid, pl.ds(n * FCH, FCH)], xbuf.at[q],
                         semld[q])

    def ld_wait(n, q):
        pltpu.make_async_copy(
            yf.at[wid, pl.ds(n * FCH, FCH)], ybuf.at[q], semld[q]).wait()
        pltpu.make_async_copy(
            xif.at[wid, pl.ds(n * FCH, FCH)], xbuf.at[q], semld[q]).wait()

    def st(n, q):
        pltpu.async_copy(obuf.at[q], ifl.at[wid, pl.ds(n * FCH, FCH)],
                         semst[q])

    def st_wait(n, q):
        pltpu.make_async_copy(
            obuf.at[q], ifl.at[wid, pl.ds(n * FCH, FCH)], semst[q]).wait()

    def compute(q):
        @plsc.parallel_loop(0, FCH, LANES, unroll=UNROLL)
        def _(off):
            yv = ybuf[q, pl.ds(off, LANES)]
            xv = xbuf[q, pl.ds(off, LANES)]
            obuf[q, pl.ds(off, LANES)] = xv + yv * W

    ld(0, 0)
    ld(1, 1)
    ld_wait(0, 0)
    compute(0)
    ld(2, 0)
    st(0, 0)
    ld_wait(1, 1)
    compute(1)
    ld(3, 1)
    st(1, 1)

    def pair(p, carry):
        for q in (0, 1):
            n = 2 * p + q
            ld_wait(n, q)
            st_wait(n - 2, q)
            compute(q)

            @pl.when(n + 2 <= FNC - 1)
            def _():
                ld(n + 2, q)

            st(n, q)
        return carry

    lax.fori_loop(1, (FNC - 1) // 2, pair, 0)

    ld_wait(FNC - 1, 0)
    st_wait(FNC - 3, 0)
    compute(0)
    st(FNC - 1, 0)
    st_wait(FNC - 2, 1)
    st_wait(FNC - 1, 0)


def _gather_body(xf, ifl, out, t0, t1, t2, t3, ibuf, obuf,
                 sl0, sl1, ss0, ss1):
    tables = (t0, t1, t2, t3)
    semld = (sl0, sl1)
    semst = (ss0, ss1)
    cid = lax.axis_index("c")
    sid = lax.axis_index("s")
    wid = sid * 2 + cid
    b = wid // WPB
    cbase = (wid % WPB) * TPC

    for j in range(TPC):
        pltpu.sync_copy(xf.at[b, cbase + j], tables[j])

    def ld(n, q):
        pltpu.async_copy(ifl.at[b, pl.ds(n * CH, CH)], ibuf.at[q], semld[q])

    def ld_wait(n, q):
        pltpu.make_async_copy(
            ifl.at[b, pl.ds(n * CH, CH)], ibuf.at[q], semld[q]).wait()

    def out_slice(n):
        k = n // NCHUNK
        t = n % NCHUNK
        return out.at[b, k, pl.ds(cbase, TPC), pl.ds(t * CH, CH)]

    def st(n, q):
        pltpu.async_copy(obuf.at[q], out_slice(n), semst[q])

    def st_wait(n, q):
        pltpu.make_async_copy(obuf.at[q], out_slice(n), semst[q]).wait()

    def compute(q):
        @plsc.parallel_loop(0, CH, LANES, unroll=UNROLL)
        def _(off):
            iv = ibuf[q, pl.ds(off, LANES)]
            for j in range(TPC):
                obuf[q, j, pl.ds(off, LANES)] = plsc.load_gather(
                    tables[j], [iv])

    ld(0, 0)
    ld(1, 1)
    ld_wait(0, 0)
    compute(0)
    ld(2, 0)
    st(0, 0)
    ld_wait(1, 1)
    compute(1)
    ld(3, 1)
    st(1, 1)

    def pair(p, carry):
        for q in (0, 1):
            n = 2 * p + q
            ld_wait(n, q)
            st_wait(n - 2, q)
            compute(q)

            @pl.when(n + 2 <= NTOT - 1)
            def _():
                ld(n + 2, q)

            st(n, q)
        return carry

    lax.fori_loop(1, (NTOT - 1) // 2, pair, 0)

    ld_wait(NTOT - 1, 0)
    st_wait(NTOT - 3, 0)
    compute(0)
    st(NTOT - 1, 0)
    st_wait(NTOT - 2, 1)
    st_wait(NTOT - 1, 0)


@jax.jit
def kernel(x, inref_y, inref_x):
    xf = x.reshape(B, C, HW)
    yflat = inref_y.reshape(NW, FPW)
    xiflat = inref_x.reshape(NW, FPW)
    mesh = plsc.VectorSubcoreMesh(core_axis_name="c", subcore_axis_name="s")
    sc_params = pltpu.CompilerParams(needs_layout_passes=False)

    ifl = pl.kernel(
        _flatten_body,
        out_type=jax.ShapeDtypeStruct((NW, FPW), jnp.int32),
        mesh=mesh,
        compiler_params=sc_params,
        scratch_types=[
            pltpu.VMEM((2, FCH), jnp.int32),
            pltpu.VMEM((2, FCH), jnp.int32),
            pltpu.VMEM((2, FCH), jnp.int32),
            pltpu.SemaphoreType.DMA,
            pltpu.SemaphoreType.DMA,
            pltpu.SemaphoreType.DMA,
            pltpu.SemaphoreType.DMA,
        ],
    )(yflat, xiflat)

    out = pl.kernel(
        _gather_body,
        out_type=jax.ShapeDtypeStruct((B, K, C, M), jnp.float32),
        mesh=mesh,
        compiler_params=sc_params,
        scratch_types=[
            pltpu.VMEM((HW,), jnp.float32),
            pltpu.VMEM((HW,), jnp.float32),
            pltpu.VMEM((HW,), jnp.float32),
            pltpu.VMEM((HW,), jnp.float32),
            pltpu.VMEM((2, CH), jnp.int32),
            pltpu.VMEM((2, TPC, CH), jnp.float32),
            pltpu.SemaphoreType.DMA,
            pltpu.SemaphoreType.DMA,
            pltpu.SemaphoreType.DMA,
            pltpu.SemaphoreType.DMA,
        ],
    )(xf, ifl.reshape(B, J))
    return out


# speedup vs baseline: 4872.1326x; 1.0154x over previous
import jax
import jax.numpy as jnp
from jax import lax
from jax.experimental import pallas as pl
from jax.experimental.pallas import tpu as pltpu
from jax.experimental.pallas import tpu_sc as plsc

B, C, H, W = 4, 32, 128, 128
HW = H * W
K = 9
M = 9 * 64 * 64
J = K * M
CH = 4096
NCHUNK = M // CH
NTOT = K * NCHUNK
LANES = 16
TPC = 4
NW = 32
WPB = NW // B
UNROLL = 8

FTOT = B * J
FPS = FTOT // NW
FCH = 2304
FNC = FPS // FCH
SPB = J // FPS


def _body(xf, yf, xif, out, ifl, t0, t1, t2, t3, ibuf, obuf, fybuf, fxbuf,
          fobuf, stab, sl0, sl1, ss0, ss1):
    tables = (t0, t1, t2, t3)
    semld = (sl0, sl1)
    semst = (ss0, ss1)
    cid = lax.axis_index("c")
    sid = lax.axis_index("s")
    b = cid * 2 + sid // 8
    cbase = (sid % 8) * TPC

    for j in range(TPC):
        pltpu.async_copy(xf.at[b, cbase + j], tables[j], stab)

    r = cid * 16 + sid
    fb = r // SPB
    foff = (r % SPB) * FPS

    def fld(n, q):
        pltpu.async_copy(yf.at[r, pl.ds(n * FCH, FCH)], fybuf.at[q],
                         semld[q])
        pltpu.async_copy(xif.at[r, pl.ds(n * FCH, FCH)], fxbuf.at[q],
                         semld[q])

    def fld_wait(n, q):
        pltpu.make_async_copy(
            yf.at[r, pl.ds(n * FCH, FCH)], fybuf.at[q], semld[q]).wait()
        pltpu.make_async_copy(
            xif.at[r, pl.ds(n * FCH, FCH)], fxbuf.at[q], semld[q]).wait()

    def fst(n, q):
        pltpu.async_copy(fobuf.at[q],
                         ifl.at[fb, pl.ds(foff + n * FCH, FCH)], semst[q])

    def fst_wait(n, q):
        pltpu.make_async_copy(
            fobuf.at[q], ifl.at[fb, pl.ds(foff + n * FCH, FCH)],
            semst[q]).wait()

    def fcompute(q):
        @plsc.parallel_loop(0, FCH, LANES, unroll=UNROLL)
        def _(off):
            yv = fybuf[q, pl.ds(off, LANES)]
            xv = fxbuf[q, pl.ds(off, LANES)]
            fobuf[q, pl.ds(off, LANES)] = xv + yv * W

    fld(0, 0)
    fld(1, 1)
    fld_wait(0, 0)
    fcompute(0)
    fld(2, 0)
    fst(0, 0)
    fld_wait(1, 1)
    fcompute(1)
    fld(3, 1)
    fst(1, 1)

    def fpair(p, carry):
        for q in (0, 1):
            n = 2 * p + q
            fld_wait(n, q)
            fst_wait(n - 2, q)
            fcompute(q)

            @pl.when(n + 2 <= FNC - 1)
            def _():
                fld(n + 2, q)

            fst(n, q)
        return carry

    lax.fori_loop(1, FNC // 2, fpair, 0)

    fst_wait(FNC - 2, 0)
    fst_wait(FNC - 1, 1)

    for j in range(TPC):
        pltpu.make_async_copy(xf.at[b, cbase + j], tables[j], stab).wait()
    plsc.subcore_barrier()

    def ld(n, q):
        pltpu.async_copy(ifl.at[b, pl.ds(n * CH, CH)], ibuf.at[q], semld[q])

    def ld_wait(n, q):
        pltpu.make_async_copy(
            ifl.at[b, pl.ds(n * CH, CH)], ibuf.at[q], semld[q]).wait()

    def out_slice(n):
        k = n // NCHUNK
        t = n % NCHUNK
        return out.at[b, k, pl.ds(cbase, TPC), pl.ds(t * CH, CH)]

    def st(n, q):
        pltpu.async_copy(obuf.at[q], out_slice(n), semst[q])

    def st_wait(n, q):
        pltpu.make_async_copy(obuf.at[q], out_slice(n), semst[q]).wait()

    def compute(q):
        @plsc.parallel_loop(0, CH, LANES, unroll=UNROLL)
        def _(off):
            iv = ibuf[q, pl.ds(off, LANES)]
            for j in range(TPC):
                obuf[q, j, pl.ds(off, LANES)] = plsc.load_gather(
                    tables[j], [iv])

    ld(0, 0)
    ld(1, 1)
    ld_wait(0, 0)
    compute(0)
    ld(2, 0)
    st(0, 0)
    ld_wait(1, 1)
    compute(1)
    ld(3, 1)
    st(1, 1)

    def pair(p, carry):
        for q in (0, 1):
            n = 2 * p + q
            ld_wait(n, q)
            st_wait(n - 2, q)
            compute(q)

            @pl.when(n + 2 <= NTOT - 1)
            def _():
                ld(n + 2, q)

            st(n, q)
        return carry

    lax.fori_loop(1, (NTOT - 1) // 2, pair, 0)

    ld_wait(NTOT - 1, 0)
    st_wait(NTOT - 3, 0)
    compute(0)
    st(NTOT - 1, 0)
    st_wait(NTOT - 2, 1)
    st_wait(NTOT - 1, 0)


@jax.jit
def kernel(x, inref_y, inref_x):
    xf = x.reshape(B, C, HW)
    yflat = inref_y.reshape(NW, FPS)
    xiflat = inref_x.reshape(NW, FPS)
    mesh = plsc.VectorSubcoreMesh(core_axis_name="c", subcore_axis_name="s")
    out, _ = pl.kernel(
        _body,
        out_type=(jax.ShapeDtypeStruct((B, K, C, M), jnp.float32),
                  jax.ShapeDtypeStruct((B, J), jnp.int32)),
        mesh=mesh,
        compiler_params=pltpu.CompilerParams(needs_layout_passes=False),
        scratch_types=[
            pltpu.VMEM((HW,), jnp.float32),
            pltpu.VMEM((HW,), jnp.float32),
            pltpu.VMEM((HW,), jnp.float32),
            pltpu.VMEM((HW,), jnp.float32),
            pltpu.VMEM((2, CH), jnp.int32),
            pltpu.VMEM((2, TPC, CH), jnp.float32),
            pltpu.VMEM((2, FCH), jnp.int32),
            pltpu.VMEM((2, FCH), jnp.int32),
            pltpu.VMEM((2, FCH), jnp.int32),
            pltpu.SemaphoreType.DMA,
            pltpu.SemaphoreType.DMA,
            pltpu.SemaphoreType.DMA,
            pltpu.SemaphoreType.DMA,
            pltpu.SemaphoreType.DMA,
        ],
    )(xf, yflat, xiflat)
    return out


# speedup vs baseline: 4935.3951x; 1.0130x over previous
import jax
import jax.numpy as jnp
from jax import lax
from jax.experimental import pallas as pl
from jax.experimental.pallas import tpu as pltpu
from jax.experimental.pallas import tpu_sc as plsc

B, C, H, W = 4, 32, 128, 128
HW = H * W
K = 9
M = 9 * 64 * 64
J = K * M
CH = 4608
NCHUNK = M // CH
NTOT = K * NCHUNK
LANES = 16
TPC = 4
NW = 32
WPB = NW // B
UNROLL = 8

FTOT = B * J
FPS = FTOT // NW
FCH = 2304
FNC = FPS // FCH
SPB = J // FPS


def _body(xf, yf, xif, out, ifl, t0, t1, t2, t3, ibuf, obuf, fybuf, fxbuf,
          fobuf, stab, sl0, sl1, ss0, ss1):
    tables = (t0, t1, t2, t3)
    semld = (sl0, sl1)
    semst = (ss0, ss1)
    cid = lax.axis_index("c")
    sid = lax.axis_index("s")
    b = cid * 2 + sid // 8
    cbase = (sid % 8) * TPC

    for j in range(TPC):
        pltpu.async_copy(xf.at[b, cbase + j], tables[j], stab)

    r = cid * 16 + sid
    fb = r // SPB
    foff = (r % SPB) * FPS

    def fld(n, q):
        pltpu.async_copy(yf.at[r, pl.ds(n * FCH, FCH)], fybuf.at[q],
                         semld[q])
        pltpu.async_copy(xif.at[r, pl.ds(n * FCH, FCH)], fxbuf.at[q],
                         semld[q])

    def fld_wait(n, q):
        pltpu.make_async_copy(
            yf.at[r, pl.ds(n * FCH, FCH)], fybuf.at[q], semld[q]).wait()
        pltpu.make_async_copy(
            xif.at[r, pl.ds(n * FCH, FCH)], fxbuf.at[q], semld[q]).wait()

    def fst(n, q):
        pltpu.async_copy(fobuf.at[q],
                         ifl.at[fb, pl.ds(foff + n * FCH, FCH)], semst[q])

    def fst_wait(n, q):
        pltpu.make_async_copy(
            fobuf.at[q], ifl.at[fb, pl.ds(foff + n * FCH, FCH)],
            semst[q]).wait()

    def fcompute(q):
        @plsc.parallel_loop(0, FCH, LANES, unroll=UNROLL)
        def _(off):
            yv = fybuf[q, pl.ds(off, LANES)]
            xv = fxbuf[q, pl.ds(off, LANES)]
            fobuf[q, pl.ds(off, LANES)] = xv + yv * W

    fld(0, 0)
    fld(1, 1)
    fld_wait(0, 0)
    fcompute(0)
    fld(2, 0)
    fst(0, 0)
    fld_wait(1, 1)
    fcompute(1)
    fld(3, 1)
    fst(1, 1)

    def fpair(p, carry):
        for q in (0, 1):
            n = 2 * p + q
            fld_wait(n, q)
            fst_wait(n - 2, q)
            fcompute(q)

            @pl.when(n + 2 <= FNC - 1)
            def _():
                fld(n + 2, q)

            fst(n, q)
        return carry

    lax.fori_loop(1, FNC // 2, fpair, 0)

    fst_wait(FNC - 2, 0)
    fst_wait(FNC - 1, 1)

    for j in range(TPC):
        pltpu.make_async_copy(xf.at[b, cbase + j], tables[j], stab).wait()
    plsc.subcore_barrier()

    def ld(n, q):
        pltpu.async_copy(ifl.at[b, pl.ds(n * CH, CH)], ibuf.at[q], semld[q])

    def ld_wait(n, q):
        pltpu.make_async_copy(
            ifl.at[b, pl.ds(n * CH, CH)], ibuf.at[q], semld[q]).wait()

    def out_slice(n):
        k = n // NCHUNK
        t = n % NCHUNK
        return out.at[b, k, pl.ds(cbase, TPC), pl.ds(t * CH, CH)]

    def st(n, q):
        pltpu.async_copy(obuf.at[q], out_slice(n), semst[q])

    def st_wait(n, q):
        pltpu.make_async_copy(obuf.at[q], out_slice(n), semst[q]).wait()

    def compute(q):
        @plsc.parallel_loop(0, CH, LANES, unroll=UNROLL)
        def _(off):
            iv = ibuf[q, pl.ds(off, LANES)]
            for j in range(TPC):
                obuf[q, j, pl.ds(off, LANES)] = plsc.load_gather(
                    tables[j], [iv])

    ld(0, 0)
    ld(1, 1)
    ld_wait(0, 0)
    compute(0)
    ld(2, 0)
    st(0, 0)
    ld_wait(1, 1)
    compute(1)
    ld(3, 1)
    st(1, 1)

    def pair(p, carry):
        for q in (0, 1):
            n = 2 * p + q
            ld_wait(n, q)
            st_wait(n - 2, q)
            compute(q)

            @pl.when(n + 2 <= NTOT - 1)
            def _():
                ld(n + 2, q)

            st(n, q)
        return carry

    lax.fori_loop(1, NTOT // 2, pair, 0)

    st_wait(NTOT - 2, 0)
    st_wait(NTOT - 1, 1)


@jax.jit
def kernel(x, inref_y, inref_x):
    xf = x.reshape(B, C, HW)
    yflat = inref_y.reshape(NW, FPS)
    xiflat = inref_x.reshape(NW, FPS)
    mesh = plsc.VectorSubcoreMesh(core_axis_name="c", subcore_axis_name="s")
    out, _ = pl.kernel(
        _body,
        out_type=(jax.ShapeDtypeStruct((B, K, C, M), jnp.float32),
                  jax.ShapeDtypeStruct((B, J), jnp.int32)),
        mesh=mesh,
        compiler_params=pltpu.CompilerParams(needs_layout_passes=False),
        scratch_types=[
            pltpu.VMEM((HW,), jnp.float32),
            pltpu.VMEM((HW,), jnp.float32),
            pltpu.VMEM((HW,), jnp.float32),
            pltpu.VMEM((HW,), jnp.float32),
            pltpu.VMEM((2, CH), jnp.int32),
            pltpu.VMEM((2, TPC, CH), jnp.float32),
            pltpu.VMEM((2, FCH), jnp.int32),
            pltpu.VMEM((2, FCH), jnp.int32),
            pltpu.VMEM((2, FCH), jnp.int32),
            pltpu.SemaphoreType.DMA,
            pltpu.SemaphoreType.DMA,
            pltpu.SemaphoreType.DMA,
            pltpu.SemaphoreType.DMA,
            pltpu.SemaphoreType.DMA,
        ],
    )(xf, yflat, xiflat)
    return out


# speedup vs baseline: 4941.6263x; 1.0013x over previous
import jax
import jax.numpy as jnp
from jax import lax
from jax.experimental import pallas as pl
from jax.experimental.pallas import tpu as pltpu
from jax.experimental.pallas import tpu_sc as plsc

B, C, H, W = 4, 32, 128, 128
HW = H * W
K = 9
M = 9 * 64 * 64
J = K * M
CH = 4608
NCHUNK = M // CH
NTOT = K * NCHUNK
LANES = 16
TPC = 4
NW = 32
WPB = NW // B
UNROLL = 16

FTOT = B * J
FPS = FTOT // NW
FCH = 2304
FNC = FPS // FCH
SPB = J // FPS


def _body(xf, yf, xif, out, ifl, t0, t1, t2, t3, ibuf, obuf, fybuf, fxbuf,
          fobuf, stab, sl0, sl1, ss0, ss1):
    tables = (t0, t1, t2, t3)
    semld = (sl0, sl1)
    semst = (ss0, ss1)
    cid = lax.axis_index("c")
    sid = lax.axis_index("s")
    b = cid * 2 + sid // 8
    cbase = (sid % 8) * TPC

    for j in range(TPC):
        pltpu.async_copy(xf.at[b, cbase + j], tables[j], stab)

    r = cid * 16 + sid
    fb = r // SPB
    foff = (r % SPB) * FPS

    def fld(n, q):
        pltpu.async_copy(yf.at[r, pl.ds(n * FCH, FCH)], fybuf.at[q],
                         semld[q])
        pltpu.async_copy(xif.at[r, pl.ds(n * FCH, FCH)], fxbuf.at[q],
                         semld[q])

    def fld_wait(n, q):
        pltpu.make_async_copy(
            yf.at[r, pl.ds(n * FCH, FCH)], fybuf.at[q], semld[q]).wait()
        pltpu.make_async_copy(
            xif.at[r, pl.ds(n * FCH, FCH)], fxbuf.at[q], semld[q]).wait()

    def fst(n, q):
        pltpu.async_copy(fobuf.at[q],
                         ifl.at[fb, pl.ds(foff + n * FCH, FCH)], semst[q])

    def fst_wait(n, q):
        pltpu.make_async_copy(
            fobuf.at[q], ifl.at[fb, pl.ds(foff + n * FCH, FCH)],
            semst[q]).wait()

    def fcompute(q):
        @plsc.parallel_loop(0, FCH, LANES, unroll=UNROLL)
        def _(off):
            yv = fybuf[q, pl.ds(off, LANES)]
            xv = fxbuf[q, pl.ds(off, LANES)]
            fobuf[q, pl.ds(off, LANES)] = xv + yv * W

    fld(0, 0)
    fld(1, 1)
    fld_wait(0, 0)
    fcompute(0)
    fld(2, 0)
    fst(0, 0)
    fld_wait(1, 1)
    fcompute(1)
    fld(3, 1)
    fst(1, 1)

    def fpair(p, carry):
        for q in (0, 1):
            n = 2 * p + q
            fld_wait(n, q)
            fst_wait(n - 2, q)
            fcompute(q)

            @pl.when(n + 2 <= FNC - 1)
            def _():
                fld(n + 2, q)

            fst(n, q)
        return carry

    lax.fori_loop(1, FNC // 2, fpair, 0)

    fst_wait(FNC - 2, 0)
    fst_wait(FNC - 1, 1)

    for j in range(TPC):
        pltpu.make_async_copy(xf.at[b, cbase + j], tables[j], stab).wait()
    plsc.subcore_barrier()

    def ld(n, q):
        pltpu.async_copy(ifl.at[b, pl.ds(n * CH, CH)], ibuf.at[q], semld[q])

    def ld_wait(n, q):
        pltpu.make_async_copy(
            ifl.at[b, pl.ds(n * CH, CH)], ibuf.at[q], semld[q]).wait()

    def out_slice(n):
        k = n // NCHUNK
        t = n % NCHUNK
        return out.at[b, k, pl.ds(cbase, TPC), pl.ds(t * CH, CH)]

    def st(n, q):
        pltpu.async_copy(obuf.at[q], out_slice(n), semst[q])

    def st_wait(n, q):
        pltpu.make_async_copy(obuf.at[q], out_slice(n), semst[q]).wait()

    def compute(q):
        @plsc.parallel_loop(0, CH, LANES, unroll=UNROLL)
        def _(off):
            iv = ibuf[q, pl.ds(off, LANES)]
            for j in range(TPC):
                obuf[q, j, pl.ds(off, LANES)] = plsc.load_gather(
                    tables[j], [iv])

    ld(0, 0)
    ld(1, 1)
    ld_wait(0, 0)
    compute(0)
    ld(2, 0)
    st(0, 0)
    ld_wait(1, 1)
    compute(1)
    ld(3, 1)
    st(1, 1)

    def pair(p, carry):
        for q in (0, 1):
            n = 2 * p + q
            ld_wait(n, q)
            st_wait(n - 2, q)
            compute(q)

            @pl.when(n + 2 <= NTOT - 1)
            def _():
                ld(n + 2, q)

            st(n, q)
        return carry

    lax.fori_loop(1, NTOT // 2, pair, 0)

    st_wait(NTOT - 2, 0)
    st_wait(NTOT - 1, 1)


@jax.jit
def kernel(x, inref_y, inref_x):
    xf = x.reshape(B, C, HW)
    yflat = inref_y.reshape(NW, FPS)
    xiflat = inref_x.reshape(NW, FPS)
    mesh = plsc.VectorSubcoreMesh(core_axis_name="c", subcore_axis_name="s")
    out, _ = pl.kernel(
        _body,
        out_type=(jax.ShapeDtypeStruct((B, K, C, M), jnp.float32),
                  jax.ShapeDtypeStruct((B, J), jnp.int32)),
        mesh=mesh,
        compiler_params=pltpu.CompilerParams(needs_layout_passes=False),
        scratch_types=[
            pltpu.VMEM((HW,), jnp.float32),
            pltpu.VMEM((HW,), jnp.float32),
            pltpu.VMEM((HW,), jnp.float32),
            pltpu.VMEM((HW,), jnp.float32),
            pltpu.VMEM((2, CH), jnp.int32),
            pltpu.VMEM((2, TPC, CH), jnp.float32),
            pltpu.VMEM((2, FCH), jnp.int32),
            pltpu.VMEM((2, FCH), jnp.int32),
            pltpu.VMEM((2, FCH), jnp.int32),
            pltpu.SemaphoreType.DMA,
            pltpu.SemaphoreType.DMA,
            pltpu.SemaphoreType.DMA,
            pltpu.SemaphoreType.DMA,
            pltpu.SemaphoreType.DMA,
        ],
    )(xf, yflat, xiflat)
    return out
